# Initial kernel scaffold; baseline (speedup 1.0000x reference)
import jax
import jax.numpy as jnp
from jax.experimental import pallas as pl


def kernel(x_embed, prompt_idx, prompt, prompt_key):
    raise NotImplementedError("write your pallas kernel here")


# speedup vs baseline: 1.0252x; 1.0252x over previous
import functools

import jax
import jax.numpy as jnp
from jax import lax
from jax.experimental import pallas as pl
from jax.experimental.pallas import tpu as pltpu
from jax.experimental.pallas import tpu_sc as plsc

NL, POOL, LEN, ED = 12, 64, 5, 2048
BS, TOPK = 128, 2

NC, NS, L = 2, 16, 16
NW = NC * NS
QS = 4
DQ = (LEN * ED) // QS
NQ = BS * TOPK * QS
RQ = NL * NQ
QPW = RQ // NW
NCH = QPW // L
JMASK = BS * TOPK - 1


def _gather_body(tableq, idxf, outq, idxv, idxc, buf0, buf1, sem0, sem1):
    wid = lax.axis_index("s") * NC + lax.axis_index("c")
    base = wid * QPW

    pltpu.sync_copy(idxf, idxv)

    iota = lax.iota(jnp.int32, L)
    for g in range(NCH):
        q = base + g * L + iota
        r = q >> 2
        p = q & 3
        nl = r >> 8
        jj = r & JMASK
        val = plsc.load_gather(idxv, [jj])
        idxc[g, :] = (((nl << 6) + val) << 2) | p

    bufs = (buf0, buf1)
    sems = (sem0, sem1)
    for b in range(2):
        pltpu.make_async_copy(tableq.at[idxc.at[b]], bufs[b], sems[b]).start()
    for g in range(NCH):
        b = g & 1
        pltpu.make_async_copy(tableq.at[idxc.at[g]], bufs[b], sems[b]).wait()
        pltpu.sync_copy(bufs[b], outq.at[pl.ds(base + g * L, L)])
        if g + 2 < NCH:
            pltpu.make_async_copy(
                tableq.at[idxc.at[g + 2]], bufs[b], sems[b]
            ).start()


@functools.partial(
    pl.kernel,
    out_type=jax.ShapeDtypeStruct((RQ, DQ), jnp.float32),
    mesh=plsc.VectorSubcoreMesh(
        core_axis_name="c", subcore_axis_name="s", num_cores=NC, num_subcores=NS
    ),
    compiler_params=pltpu.CompilerParams(needs_layout_passes=False),
    scratch_types=[
        pltpu.VMEM((BS * TOPK,), jnp.int32),
        pltpu.VMEM((NCH, L), jnp.int32),
        pltpu.VMEM((L, DQ), jnp.float32),
        pltpu.VMEM((L, DQ), jnp.float32),
        pltpu.SemaphoreType.DMA,
        pltpu.SemaphoreType.DMA,
    ],
)
def _sc_gather(tableq, idxf, outq, idxv, idxc, buf0, buf1, sem0, sem1):
    _gather_body(tableq, idxf, outq, idxv, idxc, buf0, buf1, sem0, sem1)


def kernel(x_embed, prompt_idx, prompt, prompt_key):
    tableq = prompt.reshape(NL * POOL * QS, DQ)
    idxf = prompt_idx.reshape(BS * TOPK)
    outq = _sc_gather(tableq, idxf)
    return outq.reshape(NL, BS, TOPK * LEN, ED)


# speedup vs baseline: 1.0315x; 1.0061x over previous
import functools

import jax
import jax.numpy as jnp
from jax import lax
from jax.experimental import pallas as pl
from jax.experimental.pallas import tpu as pltpu
from jax.experimental.pallas import tpu_sc as plsc

NL, POOL, LEN, ED = 12, 64, 5, 2048
BS, TOPK = 128, 2

NC, NS, L = 2, 16, 16
NW = NC * NS
R = NL * BS * TOPK
RPW = R // NW
CPW = RPW // TOPK
NV = RPW // L
NBUF = 2
JMASK = BS * TOPK - 1


def _gather_body(table, idxf, out, idxv, idxc, buf0, buf1, sem0, sem1):
    wid = lax.axis_index("s") * NC + lax.axis_index("c")
    base = wid * RPW
    obase = wid * CPW

    pltpu.sync_copy(idxf, idxv)

    iota = lax.iota(jnp.int32, L)
    zeros = iota & 0
    for v in range(NV):
        p = v * L + iota
        r = base + p
        nl = r >> 8
        jj = r & JMASK
        val = plsc.load_gather(idxv, [jj])
        plsc.store_scatter(idxc, [p, zeros], (nl << 6) + val)

    bufs = (buf0, buf1)
    sems = (sem0, sem1)

    def start_row(c, b):
        for t in range(TOPK):
            pltpu.make_async_copy(
                table.at[idxc.at[TOPK * c + t]],
                bufs[b].at[pl.ds(0, 1), pl.ds(t * LEN, LEN), :],
                sems[b],
            ).start()

    def wait_row(c, b):
        pltpu.make_async_copy(out.at[pl.ds(obase, 1)], bufs[b], sems[b]).wait()

    def scatter_row(c, b):
        pltpu.sync_copy(bufs[b], out.at[pl.ds(obase + c, 1)])

    for b in range(NBUF):
        start_row(b, b)

    def body(g, _):
        for b in range(NBUF):
            c = g * NBUF + b
            wait_row(c, b)
            scatter_row(c, b)
            start_row(c + NBUF, b)
        return None

    lax.fori_loop(0, CPW // NBUF - 1, body, None)

    for b in range(NBUF):
        c = CPW - NBUF + b
        wait_row(c, b)
        scatter_row(c, b)


@functools.partial(
    pl.kernel,
    out_type=jax.ShapeDtypeStruct((NL * BS, TOPK * LEN, ED), jnp.float32),
    mesh=plsc.VectorSubcoreMesh(
        core_axis_name="c", subcore_axis_name="s", num_cores=NC, num_subcores=NS
    ),
    compiler_params=pltpu.CompilerParams(
        needs_layout_passes=False, use_tc_tiling_on_sc=False
    ),
    scratch_types=[
        pltpu.VMEM((BS * TOPK,), jnp.int32),
        pltpu.VMEM((RPW, 1), jnp.int32),
        pltpu.VMEM((1, TOPK * LEN, ED), jnp.float32),
        pltpu.VMEM((1, TOPK * LEN, ED), jnp.float32),
        pltpu.SemaphoreType.DMA,
        pltpu.SemaphoreType.DMA,
    ],
)
def _sc_gather(table, idxf, out, idxv, idxc, buf0, buf1, sem0, sem1):
    _gather_body(table, idxf, out, idxv, idxc, buf0, buf1, sem0, sem1)


def kernel(x_embed, prompt_idx, prompt, prompt_key):
    table = prompt.reshape(NL * POOL, LEN, ED)
    idxf = prompt_idx.reshape(BS * TOPK)
    out = _sc_gather(table, idxf)
    return out.reshape(NL, BS, TOPK * LEN, ED)


# speedup vs baseline: 3.9383x; 3.8180x over previous
import functools

import jax
import jax.numpy as jnp
from jax import lax
from jax.experimental import pallas as pl
from jax.experimental.pallas import tpu as pltpu
from jax.experimental.pallas import tpu_sc as plsc

NL, POOL, LEN, ED = 12, 64, 5, 2048
BS, TOPK = 128, 2

NC, NS, L = 2, 16, 16
NW = NC * NS
SL, LN = 8, 128
ET = ED // LN
TR = NL * LEN * POOL * ET
OR = NL * TOPK * LEN * BS * ET
RPW = OR // NW
CHUNK = 128
NCH = RPW // CHUNK
VPC = CHUNK // L
NBUF = 2


def _gather_body(table, idxf, out, idxv, idxc, buf0, buf1, sem0, sem1):
    wid = lax.axis_index("s") * NC + lax.axis_index("c")
    obase = wid * RPW

    pltpu.sync_copy(idxf, idxv)

    iota = lax.iota(jnp.int32, L)

    def fill_chunk(c):
        for i in range(VPC):
            o = obase + c * CHUNK + i * L + iota
            nlr = o >> 11
            nl = (nlr * 205) >> 11
            r = nlr - nl * 10
            rem = o & 2047
            bt = rem >> 7
            et = (rem >> 3) & 15
            bi = rem & 7
            b = (bt << 3) | bi
            t = (r * 52) >> 8
            l = r - t * 5
            p = plsc.load_gather(idxv, [(b << 1) | t])
            row = (((nl * 5 + l) << 10) | ((p >> 3) << 7)
                   | (et << 3) | (p & 7))
            idxc[c, pl.ds(i * L, L)] = row

    bufs = (buf0, buf1)
    sems = (sem0, sem1)

    def start(c, b):
        pltpu.make_async_copy(table.at[idxc.at[c]], bufs[b], sems[b]).start()

    def wait(b):
        pltpu.make_async_copy(
            table.at[idxc.at[0]], bufs[b], sems[b]
        ).wait()

    def scatter(c, b):
        pltpu.sync_copy(bufs[b], out.at[pl.ds(obase + c * CHUNK, CHUNK)])

    for b in range(NBUF):
        fill_chunk(b)
        start(b, b)

    def body(g, _):
        for b in range(NBUF):
            c = g * NBUF + b
            wait(b)
            scatter(c, b)
            fill_chunk(c + NBUF)
            start(c + NBUF, b)
        return None

    lax.fori_loop(0, NCH // NBUF - 1, body, None)

    for b in range(NBUF):
        c = NCH - NBUF + b
        wait(b)
        scatter(c, b)


@functools.partial(
    pl.kernel,
    out_type=jax.ShapeDtypeStruct((OR, LN), jnp.float32),
    mesh=plsc.VectorSubcoreMesh(
        core_axis_name="c", subcore_axis_name="s", num_cores=NC, num_subcores=NS
    ),
    compiler_params=pltpu.CompilerParams(
        needs_layout_passes=False, use_tc_tiling_on_sc=False
    ),
    scratch_types=[
        pltpu.VMEM((BS * TOPK,), jnp.int32),
        pltpu.VMEM((NCH, CHUNK), jnp.int32),
        pltpu.VMEM((CHUNK, LN), jnp.float32),
        pltpu.VMEM((CHUNK, LN), jnp.float32),
        pltpu.SemaphoreType.DMA,
        pltpu.SemaphoreType.DMA,
    ],
)
def _sc_gather(table, idxf, out, idxv, idxc, buf0, buf1, sem0, sem1):
    _gather_body(table, idxf, out, idxv, idxc, buf0, buf1, sem0, sem1)


def kernel(x_embed, prompt_idx, prompt, prompt_key):
    table = (
        prompt.reshape(NL, POOL // SL, SL, LEN, ET, LN)
        .transpose(0, 3, 1, 4, 2, 5)
        .reshape(TR, LN)
    )
    idxf = prompt_idx.reshape(BS * TOPK)
    out = _sc_gather(table, idxf)
    return (
        out.reshape(NL, TOPK * LEN, BS // SL, ET, SL, LN)
        .transpose(0, 2, 4, 1, 3, 5)
        .reshape(NL, BS, TOPK * LEN, ED)
    )


# speedup vs baseline: 4.0042x; 1.0167x over previous
import functools

import jax
import jax.numpy as jnp
from jax import lax
from jax.experimental import pallas as pl
from jax.experimental.pallas import tpu as pltpu
from jax.experimental.pallas import tpu_sc as plsc

NL, POOL, LEN, ED = 12, 64, 5, 2048
BS, TOPK = 128, 2

NC, NS, L = 2, 16, 16
NW = NC * NS
SL, LN = 8, 128
ET = ED // LN
TR = NL * LEN * POOL * ET
OR = NL * TOPK * LEN * BS * ET
RPW = OR // NW
CHUNK = 128
NCH = RPW // CHUNK
VPC = CHUNK // L
NBUF = 4


def _gather_body(table, idxf, out, idxv, idxc, bufs, gsems, ssems):
    wid = lax.axis_index("s") * NC + lax.axis_index("c")
    obase = wid * RPW

    pltpu.sync_copy(idxf, idxv)

    iota = lax.iota(jnp.int32, L)

    def fill_chunk(c):
        for i in range(VPC):
            o = obase + c * CHUNK + i * L + iota
            nlr = o >> 11
            nl = (nlr * 205) >> 11
            r = nlr - nl * 10
            rem = o & 2047
            bt = rem >> 7
            et = (rem >> 3) & 15
            bi = rem & 7
            b = (bt << 3) | bi
            t = (r * 52) >> 8
            l = r - t * 5
            p = plsc.load_gather(idxv, [(b << 1) | t])
            row = (((nl * 5 + l) << 10) | ((p >> 3) << 7)
                   | (et << 3) | (p & 7))
            idxc[c, pl.ds(i * L, L)] = row

    def start_gather(c, b):
        pltpu.make_async_copy(table.at[idxc.at[c]], bufs[b], gsems[b]).start()

    def wait_gather(b):
        pltpu.make_async_copy(
            table.at[idxc.at[0]], bufs[b], gsems[b]
        ).wait()

    def start_scatter(c, b):
        pltpu.make_async_copy(
            bufs[b], out.at[pl.ds(obase + c * CHUNK, CHUNK)], ssems[b]
        ).start()

    def wait_scatter(b):
        pltpu.make_async_copy(
            bufs[b], out.at[pl.ds(obase, CHUNK)], ssems[b]
        ).wait()

    def step(c, b, refill, head):
        bg = (c + 2) % NBUF if isinstance(c, int) else (b + 2) % NBUF
        if refill:
            if not head:
                wait_scatter(bg)
            fill_chunk(c + 2)
            start_gather(c + 2, bg)
        wait_gather(b)
        start_scatter(c, b)

    for c in range(2):
        fill_chunk(c)
        start_gather(c, c)
    step(0, 0, True, True)
    step(1, 1, True, True)

    def body(g, _):
        for bb in range(NBUF):
            c = g * NBUF + 2 + bb
            step(c, (2 + bb) % NBUF, True, False)
        return None

    lax.fori_loop(0, (NCH - 2) // NBUF, body, None)
    for c in range(2 + ((NCH - 2) // NBUF) * NBUF, NCH):
        step(c, c % NBUF, False, False)

    for b in range(NBUF):
        wait_scatter(b)


@functools.partial(
    pl.kernel,
    out_type=jax.ShapeDtypeStruct((OR, LN), jnp.float32),
    mesh=plsc.VectorSubcoreMesh(
        core_axis_name="c", subcore_axis_name="s", num_cores=NC, num_subcores=NS
    ),
    compiler_params=pltpu.CompilerParams(
        needs_layout_passes=False, use_tc_tiling_on_sc=False
    ),
    scratch_types=[
        pltpu.VMEM((BS * TOPK,), jnp.int32),
        pltpu.VMEM((NCH, CHUNK), jnp.int32),
        [pltpu.VMEM((CHUNK, LN), jnp.float32) for _ in range(NBUF)],
        [pltpu.SemaphoreType.DMA for _ in range(NBUF)],
        [pltpu.SemaphoreType.DMA for _ in range(NBUF)],
    ],
)
def _sc_gather(table, idxf, out, idxv, idxc, bufs, gsems, ssems):
    _gather_body(table, idxf, out, idxv, idxc, bufs, gsems, ssems)


def kernel(x_embed, prompt_idx, prompt, prompt_key):
    table = (
        prompt.reshape(NL, POOL // SL, SL, LEN, ET, LN)
        .transpose(0, 3, 1, 4, 2, 5)
        .reshape(TR, LN)
    )
    idxf = prompt_idx.reshape(BS * TOPK)
    out = _sc_gather(table, idxf)
    return (
        out.reshape(NL, TOPK * LEN, BS // SL, ET, SL, LN)
        .transpose(0, 2, 4, 1, 3, 5)
        .reshape(NL, BS, TOPK * LEN, ED)
    )
